# baseline (speedup 1.0000x reference)
import functools

import jax
import jax.numpy as jnp
from jax import lax
from jax.experimental import pallas as pl
from jax.experimental.pallas import tpu as pltpu
from jax.experimental.pallas import tpu_sc as plsc

B = 4096
NF = 26
V = 100000
D = 16
H = 50

NC = 2
NS = 16
NW = NC * NS
L = 16

S_TOT = B * NF
V_TOT = B * H
S_PER = S_TOT // NW
V_PER = V_TOT // NW

_mesh = plsc.VectorSubcoreMesh(
    core_axis_name="c", subcore_axis_name="s", num_cores=NC, num_subcores=NS
)


@functools.partial(
    pl.kernel,
    out_type=(
        jax.ShapeDtypeStruct((S_TOT, D), jnp.float32),
        jax.ShapeDtypeStruct((V_TOT, D), jnp.float32),
    ),
    mesh=_mesh,
    compiler_params=pltpu.CompilerParams(use_tc_tiling_on_sc=False),
    scratch_types=[
        pltpu.VMEM((V_PER,), jnp.int32),
        pltpu.VMEM((V_PER, D), jnp.float32),
        pltpu.SemaphoreType.DMA,
    ],
)
def _gather_all(s_idx_hbm, v_idx_hbm, tbl_hbm, vtbl_hbm, s_out, v_out,
                idx_v, rows_v, sem):
    wid = lax.axis_index("s") * NC + lax.axis_index("c")

    sbase = wid * S_PER
    pltpu.sync_copy(s_idx_hbm.at[pl.ds(sbase, S_PER)], idx_v.at[pl.ds(0, S_PER)])

    def fixup(j, carry):
        off = j * L
        raw = idx_v[pl.ds(off, L)]
        pos = (sbase + off) + lax.iota(jnp.int32, L)
        idx_v[pl.ds(off, L)] = raw + (pos % NF) * V
        return carry

    lax.fori_loop(0, S_PER // L, fixup, 0, unroll=4)

    pltpu.async_copy(
        tbl_hbm.at[idx_v.at[pl.ds(0, S_PER)]], rows_v.at[pl.ds(0, S_PER)], sem
    ).wait()
    pltpu.sync_copy(rows_v.at[pl.ds(0, S_PER)], s_out.at[pl.ds(sbase, S_PER)])

    vbase = wid * V_PER
    pltpu.sync_copy(v_idx_hbm.at[pl.ds(vbase, V_PER)], idx_v)
    pltpu.async_copy(vtbl_hbm.at[idx_v], rows_v, sem).wait()
    pltpu.sync_copy(rows_v, v_out.at[pl.ds(vbase, V_PER)])


def kernel(sparse_idx, varlen_idx, dense_vals, sparse_tables, varlen_table):
    s_out, v_out = _gather_all(
        sparse_idx.reshape(S_TOT),
        varlen_idx.reshape(V_TOT),
        sparse_tables.reshape(NF * V, D),
        varlen_table,
    )
    return s_out.reshape(B, NF, D), v_out.reshape(B, H, D), dense_vals
